# baseline (speedup 1.0000x reference)
import functools

import jax
import jax.numpy as jnp
from jax import lax
from jax.experimental import pallas as pl
from jax.experimental.pallas import tpu as pltpu
from jax.experimental.pallas import tpu_sc as plsc

N = 10000
E = 320000
NC = 2
NS = 16
NW = NC * NS
DEGW = 16
ZLEN = 624
LASTOFF = ZLEN * (NS - 1)
LASTLEN = N - LASTOFF

_mesh = lambda: plsc.VectorSubcoreMesh(core_axis_name="c", subcore_axis_name="s")


def _worker(c, s):
  return s * NC + c


def _tilewise_copy(s, make_src, make_dst):
  r0 = pl.multiple_of(s * ZLEN, 8)

  @pl.when(s < NS - 1)
  def _():
    pltpu.sync_copy(make_src(r0, ZLEN), make_dst(r0, ZLEN))

  @pl.when(s == NS - 1)
  def _():
    pltpu.sync_copy(make_src(LASTOFF, LASTLEN), make_dst(LASTOFF, LASTLEN))


DK = 128
DNCH = E // DK
DFLOOR = DNCH // NW
DREM = DNCH % NW
DMAXN = DFLOOR + 1
DR = 6


@functools.partial(
    pl.kernel,
    out_type=jax.ShapeDtypeStruct((NC, 2, N, DEGW), jnp.float32),
    mesh=_mesh(),
    compiler_params=pltpu.CompilerParams(use_tc_tiling_on_sc=False),
    scratch_types=[
        pltpu.VMEM((DR, 2, DK), jnp.int32),
        pltpu.VMEM((DK, DEGW), jnp.float32),
        pltpu.VMEM_SHARED((N, DEGW), jnp.float32),
        pltpu.VMEM_SHARED((N, DEGW), jnp.float32),
    ] + [pltpu.SemaphoreType.DMA] * (3 * DR),
)
def _deg_kernel(eint, zrows, out, ebuf, ones_v, acc_s, acc_d, *sems):
  isem = sems[0:DR]
  ssem = sems[DR:2 * DR]
  tsem = sems[2 * DR:3 * DR]
  c = lax.axis_index("c")
  s = lax.axis_index("s")
  w = _worker(c, s)
  _tilewise_copy(s, lambda o, l: zrows.at[pl.ds(o, l)],
                 lambda o, l: acc_s.at[pl.ds(o, l)])
  _tilewise_copy(s, lambda o, l: zrows.at[pl.ds(o, l)],
                 lambda o, l: acc_d.at[pl.ds(o, l)])

  def fill(i, carry):
    ones_v[i, :] = jnp.ones((DEGW,), jnp.float32)
    return carry

  lax.fori_loop(0, DK, fill, 0)
  plsc.subcore_barrier()

  n = jnp.where(w < DREM, DFLOOR + 1, DFLOOR)

  def start_idx(t, r):
    pltpu.async_copy(eint.at[w + NW * t], ebuf.at[r], isem[r])

  def wait_idx(r):
    pltpu.make_async_copy(eint.at[0], ebuf.at[r], isem[r]).wait()

  def start_scatter(r):
    pltpu.async_copy(ones_v, acc_s.at[ebuf.at[r, 0]], ssem[r], add=True)
    pltpu.async_copy(ones_v, acc_d.at[ebuf.at[r, 1]], tsem[r], add=True)

  def wait_scatter(r):
    pltpu.make_async_copy(ones_v, acc_s.at[ebuf.at[r, 0]], ssem[r]).wait()
    pltpu.make_async_copy(ones_v, acc_d.at[ebuf.at[r, 1]], tsem[r]).wait()

  for r0 in range(3):
    @pl.when(r0 < n)
    def _():
      start_idx(r0, r0)

  def ring_body(g, carry):
    for j in range(DR):
      t = DR * g + j
      r = j

      @pl.when(t < n)
      def _():
        wait_idx(r)
        start_scatter(r)
        r3 = (j + 3) % DR

        @pl.when(t >= 3)
        def _():
          wait_scatter(r3)

        @pl.when(t + 3 < n)
        def _():
          start_idx(t + 3, r3)

    return carry

  lax.fori_loop(0, (DMAXN + DR - 1) // DR, ring_body, 0)
  for k in range(1, 4):
    for r in range(DR):
      @pl.when((n - k) % DR == r)
      def _():
        @pl.when(n >= k)
        def _():
          wait_scatter(r)

  plsc.subcore_barrier()
  _tilewise_copy(s, lambda o, l: acc_s.at[pl.ds(o, l)],
                 lambda o, l: out.at[c, 0, pl.ds(o, l)])
  _tilewise_copy(s, lambda o, l: acc_d.at[pl.ds(o, l)],
                 lambda o, l: out.at[c, 1, pl.ds(o, l)])


def _make_prop(F, K, NB, L):
  SS = max(1, K // 128)
  NCH = E // K
  FLOOR = NCH // NW
  REM = NCH % NW
  MAXN = FLOOR + (1 if REM else 0)
  R = 2 * NB
  X = 4

  @functools.partial(
      pl.kernel,
      out_type=jax.ShapeDtypeStruct((NC, N, F), jnp.float32),
      mesh=_mesh(),
      compiler_params=pltpu.CompilerParams(use_tc_tiling_on_sc=False),
      scratch_types=[
          pltpu.VMEM((R, 2, K), jnp.int32),
          pltpu.VMEM((NB, K, F), jnp.float32),
          pltpu.VMEM_SHARED((N, F), jnp.float32),
      ] + [pltpu.SemaphoreType.DMA] * (R + 2 * NB),
  )
  def _prop(x, eint, zrows, out, ebuf, rows, acc, *sems):
    isem = sems[0:R]
    gsem = sems[R:R + NB]
    ssem = sems[R + NB:R + 2 * NB]
    c = lax.axis_index("c")
    s = lax.axis_index("s")
    w = _worker(c, s)
    _tilewise_copy(s, lambda o, l: zrows.at[pl.ds(o, l)],
                   lambda o, l: acc.at[pl.ds(o, l)])
    plsc.subcore_barrier()

    n = jnp.where(w < REM, FLOOR + 1, FLOOR)

    def start_idx(t, r):
      pltpu.async_copy(eint.at[w + NW * t], ebuf.at[r], isem[r])

    def wait_idx(r):
      pltpu.make_async_copy(eint.at[0], ebuf.at[r], isem[r]).wait()

    def start_gather(r, b):
      pltpu.async_copy(x.at[ebuf.at[r, 0]], rows.at[b], gsem[b])

    def wait_gather(r, b):
      pltpu.make_async_copy(x.at[ebuf.at[r, 0]], rows.at[b], gsem[b]).wait()

    def start_scatter(r, b):
      if SS == 1:
        pltpu.async_copy(rows.at[b], acc.at[ebuf.at[r, 1]], ssem[b], add=True)
      else:
        for j in range(SS):
          pltpu.async_copy(rows.at[b, pl.ds(128 * j, 128)],
                           acc.at[ebuf.at[r, 1, pl.ds(128 * j, 128)]],
                           ssem[b], add=True)

    def wait_scatter(b):
      if SS == 1:
        pltpu.make_async_copy(rows.at[b], acc.at[ebuf.at[0, 1]],
                              ssem[b]).wait()
      else:
        for j in range(SS):
          pltpu.make_async_copy(rows.at[b, pl.ds(128 * j, 128)],
                                acc.at[ebuf.at[0, 1, pl.ds(0, 128)]],
                                ssem[b]).wait()

    for t0 in range(X):
      @pl.when(t0 < n)
      def _():
        start_idx(t0, t0)

    for t0 in range(L):
      @pl.when(t0 < n)
      def _():
        wait_idx(t0)
        start_gather(t0, t0)

    def ring_body(g, carry):
      for j in range(R):
        t = R * g + j
        r = j
        b = j % NB
        bL = (b + L) % NB
        rL = (j + L) % R

        def lead_gather():
          @pl.when(t + L < n)
          def _():
            @pl.when(t + L >= NB)
            def _():
              wait_scatter(bL)
            wait_idx(rL)
            start_gather(rL, bL)

        if L == 1:
          lead_gather()

        @pl.when(t < n)
        def _():
          wait_gather(r, b)
          start_scatter(r, b)
          if L >= 2:
            lead_gather()
          rx = (j + X) % R

          @pl.when(t + X < n)
          def _():
            start_idx(t + X, rx)

      return carry

    lax.fori_loop(0, (MAXN + R - 1) // R, ring_body, 0)
    for k in range(1, NB - L + 1):
      for b in range(NB):
        @pl.when((n - k) % NB == b)
        def _():
          @pl.when(n >= k)
          def _():
            wait_scatter(b)

    plsc.subcore_barrier()
    _tilewise_copy(s, lambda o, l: acc.at[pl.ds(o, l)],
                   lambda o, l: out.at[c, pl.ds(o, l)])

  return _prop


_prop128 = _make_prop(128, 80, 4, 2)
_prop32 = _make_prop(32, 512, 4, 2)


BT = 2000


def _norms(degp_ref):
  sdeg = degp_ref[0, 0, :, 0:1] + degp_ref[1, 0, :, 0:1]
  ddeg = degp_ref[0, 1, :, 0:1] + degp_ref[1, 1, :, 0:1]
  ns = lax.rsqrt(jnp.maximum(sdeg, 1.0))
  nd = lax.rsqrt(jnp.maximum(ddeg, 1.0))
  return ns, nd


def _tc0_body(degp_ref, f_ref, u1_ref):
  ns, _ = _norms(degp_ref)
  u1_ref[...] = f_ref[...] * ns


def _tc1_body(s1p_ref, degp_ref, w1_ref, b1_ref, w2_ref, u2_ref):
  ns, nd = _norms(degp_ref)
  s1 = (s1p_ref[0] + s1p_ref[1]) * nd
  h1 = jnp.maximum(
      jnp.dot(s1, w1_ref[...], preferred_element_type=jnp.float32) + b1_ref[...],
      0.0)
  u2_ref[...] = jnp.dot(h1 * ns, w2_ref[...], preferred_element_type=jnp.float32)


def _tc2_body(s2p_ref, degp_ref, b2_ref, w3_ref, u3_ref):
  ns, nd = _norms(degp_ref)
  h2 = jnp.maximum((s2p_ref[0] + s2p_ref[1]) * nd + b2_ref[...], 0.0)
  u3_ref[...] = jnp.dot(h2 * ns, w3_ref[...], preferred_element_type=jnp.float32)


def _tc3_body(s3p_ref, degp_ref, b3_ref, wl_ref, bl_ref, out_ref):
  _, nd = _norms(degp_ref)
  h3 = jnp.maximum((s3p_ref[0] + s3p_ref[1]) * nd + b3_ref[...], 0.0)
  out_ref[...] = (
      jnp.dot(h3, wl_ref[...], preferred_element_type=jnp.float32) + bl_ref[...])


def _degp_spec():
  return pl.BlockSpec((NC, 2, BT, DEGW), lambda i: (0, 0, i, 0))


def _full(shape):
  return pl.BlockSpec(shape, lambda i: tuple(0 for _ in shape))


def _rows_spec(F, lead=None):
  if lead is None:
    return pl.BlockSpec((BT, F), lambda i: (i, 0))
  return pl.BlockSpec((lead, BT, F), lambda i: (0, i, 0))


_GRID = (N // BT,)

_tc0 = pl.pallas_call(
    _tc0_body,
    grid=_GRID,
    in_specs=[_degp_spec(), _rows_spec(128)],
    out_specs=_rows_spec(128),
    out_shape=jax.ShapeDtypeStruct((N, 128), jnp.float32),
)

_tc1 = pl.pallas_call(
    _tc1_body,
    grid=_GRID,
    in_specs=[
        _rows_spec(128, lead=NC), _degp_spec(),
        _full((128, 512)), _full((1, 512)), _full((512, 128)),
    ],
    out_specs=_rows_spec(128),
    out_shape=jax.ShapeDtypeStruct((N, 128), jnp.float32),
)

_tc2 = pl.pallas_call(
    _tc2_body,
    grid=_GRID,
    in_specs=[
        _rows_spec(128, lead=NC), _degp_spec(),
        _full((1, 128)), _full((128, 32)),
    ],
    out_specs=_rows_spec(32),
    out_shape=jax.ShapeDtypeStruct((N, 32), jnp.float32),
)

_tc3 = pl.pallas_call(
    _tc3_body,
    grid=_GRID,
    in_specs=[
        _rows_spec(32, lead=NC), _degp_spec(),
        _full((1, 32)), _full((32, 40)), _full((1, 40)),
    ],
    out_specs=_rows_spec(40),
    out_shape=jax.ShapeDtypeStruct((N, 40), jnp.float32),
)


def kernel(features, edge_index, W1, b1, W2, b2, W3, b3, Wl, bl):
  z16 = jnp.zeros((N, DEGW), jnp.float32)
  z128 = jnp.zeros((N, 128), jnp.float32)
  z32 = jnp.zeros((N, 32), jnp.float32)
  eint128 = edge_index.reshape(2, E // 128, 128).swapaxes(0, 1)
  eint80 = edge_index.reshape(2, E // 80, 80).swapaxes(0, 1)
  eint512 = edge_index.reshape(2, E // 512, 512).swapaxes(0, 1)
  degp = _deg_kernel(eint128, z16)
  u1 = _tc0(degp, features)
  s1p = _prop128(u1, eint80, z128)
  u2 = _tc1(s1p, degp, W1, b1.reshape(1, -1), W2)
  s2p = _prop128(u2, eint80, z128)
  u3 = _tc2(s2p, degp, b2.reshape(1, -1), W3)
  s3p = _prop32(u3, eint512, z32)
  return _tc3(s3p, degp, b3.reshape(1, -1), Wl, bl.reshape(1, -1))


# speedup vs baseline: 1.0934x; 1.0934x over previous
import functools

import jax
import jax.numpy as jnp
from jax import lax
from jax.experimental import pallas as pl
from jax.experimental.pallas import tpu as pltpu
from jax.experimental.pallas import tpu_sc as plsc

N = 10000
E = 320000
NC = 2
NS = 16
NW = NC * NS
DEGW = 16
ZLEN = 624
LASTOFF = ZLEN * (NS - 1)
LASTLEN = N - LASTOFF

_mesh = lambda: plsc.VectorSubcoreMesh(core_axis_name="c", subcore_axis_name="s")


def _worker(c, s):
  return s * NC + c


def _tilewise_copy(s, make_src, make_dst):
  r0 = pl.multiple_of(s * ZLEN, 8)

  @pl.when(s < NS - 1)
  def _():
    pltpu.sync_copy(make_src(r0, ZLEN), make_dst(r0, ZLEN))

  @pl.when(s == NS - 1)
  def _():
    pltpu.sync_copy(make_src(LASTOFF, LASTLEN), make_dst(LASTOFF, LASTLEN))


DK = 128
DNCH = E // DK
DFLOOR = DNCH // NW
DREM = DNCH % NW
DMAXN = DFLOOR + 1
DR = 6


@functools.partial(
    pl.kernel,
    out_type=jax.ShapeDtypeStruct((NC, 2, N, DEGW), jnp.float32),
    mesh=_mesh(),
    compiler_params=pltpu.CompilerParams(use_tc_tiling_on_sc=False),
    scratch_types=[
        pltpu.VMEM((DR, 2, DK), jnp.int32),
        pltpu.VMEM((DK, DEGW), jnp.float32),
        pltpu.VMEM_SHARED((N, DEGW), jnp.float32),
        pltpu.VMEM_SHARED((N, DEGW), jnp.float32),
    ] + [pltpu.SemaphoreType.DMA] * (3 * DR),
)
def _deg_kernel(eint, zrows, out, ebuf, ones_v, acc_s, acc_d, *sems):
  isem = sems[0:DR]
  ssem = sems[DR:2 * DR]
  tsem = sems[2 * DR:3 * DR]
  c = lax.axis_index("c")
  s = lax.axis_index("s")
  w = _worker(c, s)
  _tilewise_copy(s, lambda o, l: zrows.at[pl.ds(o, l)],
                 lambda o, l: acc_s.at[pl.ds(o, l)])
  _tilewise_copy(s, lambda o, l: zrows.at[pl.ds(o, l)],
                 lambda o, l: acc_d.at[pl.ds(o, l)])

  def fill(i, carry):
    ones_v[i, :] = jnp.ones((DEGW,), jnp.float32)
    return carry

  lax.fori_loop(0, DK, fill, 0)
  plsc.subcore_barrier()

  n = jnp.where(w < DREM, DFLOOR + 1, DFLOOR)

  def start_idx(t, r):
    pltpu.async_copy(eint.at[w + NW * t], ebuf.at[r], isem[r])

  def wait_idx(r):
    pltpu.make_async_copy(eint.at[0], ebuf.at[r], isem[r]).wait()

  def start_scatter(r):
    pltpu.async_copy(ones_v, acc_s.at[ebuf.at[r, 0]], ssem[r], add=True)
    pltpu.async_copy(ones_v, acc_d.at[ebuf.at[r, 1]], tsem[r], add=True)

  def wait_scatter(r):
    pltpu.make_async_copy(ones_v, acc_s.at[ebuf.at[r, 0]], ssem[r]).wait()
    pltpu.make_async_copy(ones_v, acc_d.at[ebuf.at[r, 1]], tsem[r]).wait()

  for r0 in range(3):
    @pl.when(r0 < n)
    def _():
      start_idx(r0, r0)

  def ring_body(g, carry):
    for j in range(DR):
      t = DR * g + j
      r = j

      @pl.when(t < n)
      def _():
        wait_idx(r)
        start_scatter(r)
        r3 = (j + 3) % DR

        @pl.when(t >= 3)
        def _():
          wait_scatter(r3)

        @pl.when(t + 3 < n)
        def _():
          start_idx(t + 3, r3)

    return carry

  lax.fori_loop(0, (DMAXN + DR - 1) // DR, ring_body, 0)
  for k in range(1, 4):
    for r in range(DR):
      @pl.when((n - k) % DR == r)
      def _():
        @pl.when(n >= k)
        def _():
          wait_scatter(r)

  plsc.subcore_barrier()
  _tilewise_copy(s, lambda o, l: acc_s.at[pl.ds(o, l)],
                 lambda o, l: out.at[c, 0, pl.ds(o, l)])
  _tilewise_copy(s, lambda o, l: acc_d.at[pl.ds(o, l)],
                 lambda o, l: out.at[c, 1, pl.ds(o, l)])


def _make_prop(F, K, NB, L):
  SS = max(1, K // 128)
  NCH = E // K
  FLOOR = NCH // NW
  REM = NCH % NW
  MAXN = FLOOR + (1 if REM else 0)
  R = 2 * NB
  X = 4

  @functools.partial(
      pl.kernel,
      out_type=jax.ShapeDtypeStruct((NC, N, F), jnp.float32),
      mesh=_mesh(),
      compiler_params=pltpu.CompilerParams(use_tc_tiling_on_sc=False),
      scratch_types=[
          pltpu.VMEM((R, 2, K), jnp.int32),
          pltpu.VMEM((NB, K, F), jnp.float32),
          pltpu.VMEM_SHARED((N, F), jnp.float32),
      ] + [pltpu.SemaphoreType.DMA] * (R + 2 * NB),
  )
  def _prop(x, eint, zrows, out, ebuf, rows, acc, *sems):
    isem = sems[0:R]
    gsem = sems[R:R + NB]
    ssem = sems[R + NB:R + 2 * NB]
    c = lax.axis_index("c")
    s = lax.axis_index("s")
    w = _worker(c, s)
    _tilewise_copy(s, lambda o, l: zrows.at[pl.ds(o, l)],
                   lambda o, l: acc.at[pl.ds(o, l)])
    plsc.subcore_barrier()

    n = jnp.where(w < REM, FLOOR + 1, FLOOR)

    def start_idx(t, r):
      pltpu.async_copy(eint.at[w + NW * t], ebuf.at[r], isem[r])

    def wait_idx(r):
      pltpu.make_async_copy(eint.at[0], ebuf.at[r], isem[r]).wait()

    def start_gather(r, b):
      pltpu.async_copy(x.at[ebuf.at[r, 0]], rows.at[b], gsem[b])

    def wait_gather(r, b):
      pltpu.make_async_copy(x.at[ebuf.at[r, 0]], rows.at[b], gsem[b]).wait()

    def start_scatter(r, b):
      if SS == 1:
        pltpu.async_copy(rows.at[b], acc.at[ebuf.at[r, 1]], ssem[b], add=True)
      else:
        for j in range(SS):
          pltpu.async_copy(rows.at[b, pl.ds(128 * j, 128)],
                           acc.at[ebuf.at[r, 1, pl.ds(128 * j, 128)]],
                           ssem[b], add=True)

    def wait_scatter(b):
      if SS == 1:
        pltpu.make_async_copy(rows.at[b], acc.at[ebuf.at[0, 1]],
                              ssem[b]).wait()
      else:
        for j in range(SS):
          pltpu.make_async_copy(rows.at[b, pl.ds(128 * j, 128)],
                                acc.at[ebuf.at[0, 1, pl.ds(0, 128)]],
                                ssem[b]).wait()

    for t0 in range(X):
      @pl.when(t0 < n)
      def _():
        start_idx(t0, t0)

    for t0 in range(L):
      @pl.when(t0 < n)
      def _():
        wait_idx(t0)
        start_gather(t0, t0)

    def ring_body(g, carry):
      for j in range(R):
        t = R * g + j
        r = j
        b = j % NB
        bL = (b + L) % NB
        rL = (j + L) % R

        def lead_gather():
          @pl.when(t + L < n)
          def _():
            wait_idx(rL)
            start_gather(rL, bL)

        if L == 1:
          lead_gather()

        @pl.when(t < n)
        def _():
          wait_gather(r, b)
          start_scatter(r, b)

          @pl.when(t >= NB - 2)
          def _():
            wait_scatter((b + 2) % NB)

          if L >= 2:
            lead_gather()
          rx = (j + X) % R

          @pl.when(t + X < n)
          def _():
            start_idx(t + X, rx)

      return carry

    lax.fori_loop(0, (MAXN + R - 1) // R, ring_body, 0)
    for k in range(1, NB - 1):
      for b in range(NB):
        @pl.when((n - k) % NB == b)
        def _():
          @pl.when(n >= k)
          def _():
            wait_scatter(b)

    plsc.subcore_barrier()
    _tilewise_copy(s, lambda o, l: acc.at[pl.ds(o, l)],
                   lambda o, l: out.at[c, pl.ds(o, l)])

  return _prop


_prop128 = _make_prop(128, 128, 3, 1)
_prop32 = _make_prop(32, 512, 4, 2)


BT = 2000


def _norms(degp_ref):
  sdeg = degp_ref[0, 0, :, 0:1] + degp_ref[1, 0, :, 0:1]
  ddeg = degp_ref[0, 1, :, 0:1] + degp_ref[1, 1, :, 0:1]
  ns = lax.rsqrt(jnp.maximum(sdeg, 1.0))
  nd = lax.rsqrt(jnp.maximum(ddeg, 1.0))
  return ns, nd


def _tc0_body(degp_ref, f_ref, u1_ref):
  ns, _ = _norms(degp_ref)
  u1_ref[...] = f_ref[...] * ns


def _tc1_body(s1p_ref, degp_ref, w1_ref, b1_ref, w2_ref, u2_ref):
  ns, nd = _norms(degp_ref)
  s1 = (s1p_ref[0] + s1p_ref[1]) * nd
  h1 = jnp.maximum(
      jnp.dot(s1, w1_ref[...], preferred_element_type=jnp.float32) + b1_ref[...],
      0.0)
  u2_ref[...] = jnp.dot(h1 * ns, w2_ref[...], preferred_element_type=jnp.float32)


def _tc2_body(s2p_ref, degp_ref, b2_ref, w3_ref, u3_ref):
  ns, nd = _norms(degp_ref)
  h2 = jnp.maximum((s2p_ref[0] + s2p_ref[1]) * nd + b2_ref[...], 0.0)
  u3_ref[...] = jnp.dot(h2 * ns, w3_ref[...], preferred_element_type=jnp.float32)


def _tc3_body(s3p_ref, degp_ref, b3_ref, wl_ref, bl_ref, out_ref):
  _, nd = _norms(degp_ref)
  h3 = jnp.maximum((s3p_ref[0] + s3p_ref[1]) * nd + b3_ref[...], 0.0)
  out_ref[...] = (
      jnp.dot(h3, wl_ref[...], preferred_element_type=jnp.float32) + bl_ref[...])


def _degp_spec():
  return pl.BlockSpec((NC, 2, BT, DEGW), lambda i: (0, 0, i, 0))


def _full(shape):
  return pl.BlockSpec(shape, lambda i: tuple(0 for _ in shape))


def _rows_spec(F, lead=None):
  if lead is None:
    return pl.BlockSpec((BT, F), lambda i: (i, 0))
  return pl.BlockSpec((lead, BT, F), lambda i: (0, i, 0))


_GRID = (N // BT,)

_tc0 = pl.pallas_call(
    _tc0_body,
    grid=_GRID,
    in_specs=[_degp_spec(), _rows_spec(128)],
    out_specs=_rows_spec(128),
    out_shape=jax.ShapeDtypeStruct((N, 128), jnp.float32),
)

_tc1 = pl.pallas_call(
    _tc1_body,
    grid=_GRID,
    in_specs=[
        _rows_spec(128, lead=NC), _degp_spec(),
        _full((128, 512)), _full((1, 512)), _full((512, 128)),
    ],
    out_specs=_rows_spec(128),
    out_shape=jax.ShapeDtypeStruct((N, 128), jnp.float32),
)

_tc2 = pl.pallas_call(
    _tc2_body,
    grid=_GRID,
    in_specs=[
        _rows_spec(128, lead=NC), _degp_spec(),
        _full((1, 128)), _full((128, 32)),
    ],
    out_specs=_rows_spec(32),
    out_shape=jax.ShapeDtypeStruct((N, 32), jnp.float32),
)

_tc3 = pl.pallas_call(
    _tc3_body,
    grid=_GRID,
    in_specs=[
        _rows_spec(32, lead=NC), _degp_spec(),
        _full((1, 32)), _full((32, 40)), _full((1, 40)),
    ],
    out_specs=_rows_spec(40),
    out_shape=jax.ShapeDtypeStruct((N, 40), jnp.float32),
)


def kernel(features, edge_index, W1, b1, W2, b2, W3, b3, Wl, bl):
  z16 = jnp.zeros((N, DEGW), jnp.float32)
  z128 = jnp.zeros((N, 128), jnp.float32)
  z32 = jnp.zeros((N, 32), jnp.float32)
  eint128 = edge_index.reshape(2, E // 128, 128).swapaxes(0, 1)
  eint512 = edge_index.reshape(2, E // 512, 512).swapaxes(0, 1)
  degp = _deg_kernel(eint128, z16)
  u1 = _tc0(degp, features)
  s1p = _prop128(u1, eint128, z128)
  u2 = _tc1(s1p, degp, W1, b1.reshape(1, -1), W2)
  s2p = _prop128(u2, eint128, z128)
  u3 = _tc2(s2p, degp, b2.reshape(1, -1), W3)
  s3p = _prop32(u3, eint512, z32)
  return _tc3(s3p, degp, b3.reshape(1, -1), Wl, bl.reshape(1, -1))


# speedup vs baseline: 1.0952x; 1.0016x over previous
import functools

import jax
import jax.numpy as jnp
from jax import lax
from jax.experimental import pallas as pl
from jax.experimental.pallas import tpu as pltpu
from jax.experimental.pallas import tpu_sc as plsc

N = 10000
E = 320000
NC = 2
NS = 16
NW = NC * NS
DEGW = 16
ZLEN = 624
LASTOFF = ZLEN * (NS - 1)
LASTLEN = N - LASTOFF

_mesh = lambda: plsc.VectorSubcoreMesh(core_axis_name="c", subcore_axis_name="s")


def _worker(c, s):
  return s * NC + c


def _tilewise_copy(s, make_src, make_dst):
  r0 = pl.multiple_of(s * ZLEN, 8)

  @pl.when(s < NS - 1)
  def _():
    pltpu.sync_copy(make_src(r0, ZLEN), make_dst(r0, ZLEN))

  @pl.when(s == NS - 1)
  def _():
    pltpu.sync_copy(make_src(LASTOFF, LASTLEN), make_dst(LASTOFF, LASTLEN))


DK = 128
DNCH = E // DK
DFLOOR = DNCH // NW
DREM = DNCH % NW
DMAXN = DFLOOR + 1
DR = 6


@functools.partial(
    pl.kernel,
    out_type=jax.ShapeDtypeStruct((NC, 2, N, DEGW), jnp.float32),
    mesh=_mesh(),
    compiler_params=pltpu.CompilerParams(use_tc_tiling_on_sc=False),
    scratch_types=[
        pltpu.VMEM((DR, 2, DK), jnp.int32),
        pltpu.VMEM((DK, DEGW), jnp.float32),
        pltpu.VMEM_SHARED((N, DEGW), jnp.float32),
        pltpu.VMEM_SHARED((N, DEGW), jnp.float32),
    ] + [pltpu.SemaphoreType.DMA] * (3 * DR),
)
def _deg_kernel(eint, zrows, out, ebuf, ones_v, acc_s, acc_d, *sems):
  isem = sems[0:DR]
  ssem = sems[DR:2 * DR]
  tsem = sems[2 * DR:3 * DR]
  c = lax.axis_index("c")
  s = lax.axis_index("s")
  w = _worker(c, s)
  _tilewise_copy(s, lambda o, l: zrows.at[pl.ds(o, l)],
                 lambda o, l: acc_s.at[pl.ds(o, l)])
  _tilewise_copy(s, lambda o, l: zrows.at[pl.ds(o, l)],
                 lambda o, l: acc_d.at[pl.ds(o, l)])

  def fill(i, carry):
    ones_v[i, :] = jnp.ones((DEGW,), jnp.float32)
    return carry

  lax.fori_loop(0, DK, fill, 0)
  plsc.subcore_barrier()

  n = jnp.where(w < DREM, DFLOOR + 1, DFLOOR)

  def start_idx(t, r):
    pltpu.async_copy(eint.at[w + NW * t], ebuf.at[r], isem[r])

  def wait_idx(r):
    pltpu.make_async_copy(eint.at[0], ebuf.at[r], isem[r]).wait()

  def start_scatter(r):
    pltpu.async_copy(ones_v, acc_s.at[ebuf.at[r, 0]], ssem[r], add=True)
    pltpu.async_copy(ones_v, acc_d.at[ebuf.at[r, 1]], tsem[r], add=True)

  def wait_scatter(r):
    pltpu.make_async_copy(ones_v, acc_s.at[ebuf.at[r, 0]], ssem[r]).wait()
    pltpu.make_async_copy(ones_v, acc_d.at[ebuf.at[r, 1]], tsem[r]).wait()

  for r0 in range(3):
    @pl.when(r0 < n)
    def _():
      start_idx(r0, r0)

  def ring_body(g, carry):
    for j in range(DR):
      t = DR * g + j
      r = j

      @pl.when(t < n)
      def _():
        wait_idx(r)
        start_scatter(r)
        r3 = (j + 3) % DR

        @pl.when(t >= 3)
        def _():
          wait_scatter(r3)

        @pl.when(t + 3 < n)
        def _():
          start_idx(t + 3, r3)

    return carry

  lax.fori_loop(0, (DMAXN + DR - 1) // DR, ring_body, 0)
  for k in range(1, 4):
    for r in range(DR):
      @pl.when((n - k) % DR == r)
      def _():
        @pl.when(n >= k)
        def _():
          wait_scatter(r)

  plsc.subcore_barrier()
  _tilewise_copy(s, lambda o, l: acc_s.at[pl.ds(o, l)],
                 lambda o, l: out.at[c, 0, pl.ds(o, l)])
  _tilewise_copy(s, lambda o, l: acc_d.at[pl.ds(o, l)],
                 lambda o, l: out.at[c, 1, pl.ds(o, l)])


def _make_prop(F, K, NB, L):
  SS = max(1, K // 128)
  NCH = E // K
  FLOOR = NCH // NW
  REM = NCH % NW
  MAXN = FLOOR + (1 if REM else 0)
  R = 2 * NB
  X = 4

  @functools.partial(
      pl.kernel,
      out_type=jax.ShapeDtypeStruct((NC, N, F), jnp.float32),
      mesh=_mesh(),
      compiler_params=pltpu.CompilerParams(use_tc_tiling_on_sc=False),
      scratch_types=[
          pltpu.VMEM((R, 2, K), jnp.int32),
          pltpu.VMEM((NB, K, F), jnp.float32),
          pltpu.VMEM_SHARED((N, F), jnp.float32),
      ] + [pltpu.SemaphoreType.DMA] * (R + 2 * NB),
  )
  def _prop(x, eint, zrows, out, ebuf, rows, acc, *sems):
    isem = sems[0:R]
    gsem = sems[R:R + NB]
    ssem = sems[R + NB:R + 2 * NB]
    c = lax.axis_index("c")
    s = lax.axis_index("s")
    w = _worker(c, s)
    _tilewise_copy(s, lambda o, l: zrows.at[pl.ds(o, l)],
                   lambda o, l: acc.at[pl.ds(o, l)])
    plsc.subcore_barrier()

    n = jnp.where(w < REM, FLOOR + 1, FLOOR)

    def start_idx(t, r):
      pltpu.async_copy(eint.at[w + NW * t], ebuf.at[r], isem[r])

    def wait_idx(r):
      pltpu.make_async_copy(eint.at[0], ebuf.at[r], isem[r]).wait()

    def start_gather(r, b):
      pltpu.async_copy(x.at[ebuf.at[r, 0]], rows.at[b], gsem[b])

    def wait_gather(r, b):
      pltpu.make_async_copy(x.at[ebuf.at[r, 0]], rows.at[b], gsem[b]).wait()

    def start_scatter(r, b):
      if SS == 1:
        pltpu.async_copy(rows.at[b], acc.at[ebuf.at[r, 1]], ssem[b], add=True)
      else:
        for j in range(SS):
          pltpu.async_copy(rows.at[b, pl.ds(128 * j, 128)],
                           acc.at[ebuf.at[r, 1, pl.ds(128 * j, 128)]],
                           ssem[b], add=True)

    def wait_scatter(b):
      if SS == 1:
        pltpu.make_async_copy(rows.at[b], acc.at[ebuf.at[0, 1]],
                              ssem[b]).wait()
      else:
        for j in range(SS):
          pltpu.make_async_copy(rows.at[b, pl.ds(128 * j, 128)],
                                acc.at[ebuf.at[0, 1, pl.ds(0, 128)]],
                                ssem[b]).wait()

    for t0 in range(X):
      @pl.when(t0 < n)
      def _():
        start_idx(t0, t0)

    for t0 in range(L):
      @pl.when(t0 < n)
      def _():
        wait_idx(t0)
        start_gather(t0, t0)

    def ring_body(g, carry):
      for j in range(R):
        t = R * g + j
        r = j
        b = j % NB
        bL = (b + L) % NB
        rL = (j + L) % R

        def lead_gather():
          @pl.when(t + L < n)
          def _():
            wait_idx(rL)
            start_gather(rL, bL)

        if L == 1:
          lead_gather()

        @pl.when(t < n)
        def _():
          wait_gather(r, b)
          start_scatter(r, b)

          @pl.when(t >= NB - 2)
          def _():
            wait_scatter((b + 2) % NB)

          if L >= 2:
            lead_gather()
          rx = (j + X) % R

          @pl.when(t + X < n)
          def _():
            start_idx(t + X, rx)

      return carry

    lax.fori_loop(0, (MAXN + R - 1) // R, ring_body, 0)
    for k in range(1, NB - 1):
      for b in range(NB):
        @pl.when((n - k) % NB == b)
        def _():
          @pl.when(n >= k)
          def _():
            wait_scatter(b)

    plsc.subcore_barrier()
    _tilewise_copy(s, lambda o, l: acc.at[pl.ds(o, l)],
                   lambda o, l: out.at[c, pl.ds(o, l)])

  return _prop


_prop128 = _make_prop(128, 128, 3, 2)
_prop32 = _make_prop(32, 512, 4, 2)


BT = 2000


def _norms(degp_ref):
  sdeg = degp_ref[0, 0, :, 0:1] + degp_ref[1, 0, :, 0:1]
  ddeg = degp_ref[0, 1, :, 0:1] + degp_ref[1, 1, :, 0:1]
  ns = lax.rsqrt(jnp.maximum(sdeg, 1.0))
  nd = lax.rsqrt(jnp.maximum(ddeg, 1.0))
  return ns, nd


def _tc0_body(degp_ref, f_ref, u1_ref):
  ns, _ = _norms(degp_ref)
  u1_ref[...] = f_ref[...] * ns


def _tc1_body(s1p_ref, degp_ref, w1_ref, b1_ref, w2_ref, u2_ref):
  ns, nd = _norms(degp_ref)
  s1 = (s1p_ref[0] + s1p_ref[1]) * nd
  h1 = jnp.maximum(
      jnp.dot(s1, w1_ref[...], preferred_element_type=jnp.float32) + b1_ref[...],
      0.0)
  u2_ref[...] = jnp.dot(h1 * ns, w2_ref[...], preferred_element_type=jnp.float32)


def _tc2_body(s2p_ref, degp_ref, b2_ref, w3_ref, u3_ref):
  ns, nd = _norms(degp_ref)
  h2 = jnp.maximum((s2p_ref[0] + s2p_ref[1]) * nd + b2_ref[...], 0.0)
  u3_ref[...] = jnp.dot(h2 * ns, w3_ref[...], preferred_element_type=jnp.float32)


def _tc3_body(s3p_ref, degp_ref, b3_ref, wl_ref, bl_ref, out_ref):
  _, nd = _norms(degp_ref)
  h3 = jnp.maximum((s3p_ref[0] + s3p_ref[1]) * nd + b3_ref[...], 0.0)
  out_ref[...] = (
      jnp.dot(h3, wl_ref[...], preferred_element_type=jnp.float32) + bl_ref[...])


def _degp_spec():
  return pl.BlockSpec((NC, 2, BT, DEGW), lambda i: (0, 0, i, 0))


def _full(shape):
  return pl.BlockSpec(shape, lambda i: tuple(0 for _ in shape))


def _rows_spec(F, lead=None):
  if lead is None:
    return pl.BlockSpec((BT, F), lambda i: (i, 0))
  return pl.BlockSpec((lead, BT, F), lambda i: (0, i, 0))


_GRID = (N // BT,)

_tc0 = pl.pallas_call(
    _tc0_body,
    grid=_GRID,
    in_specs=[_degp_spec(), _rows_spec(128)],
    out_specs=_rows_spec(128),
    out_shape=jax.ShapeDtypeStruct((N, 128), jnp.float32),
)

_tc1 = pl.pallas_call(
    _tc1_body,
    grid=_GRID,
    in_specs=[
        _rows_spec(128, lead=NC), _degp_spec(),
        _full((128, 512)), _full((1, 512)), _full((512, 128)),
    ],
    out_specs=_rows_spec(128),
    out_shape=jax.ShapeDtypeStruct((N, 128), jnp.float32),
)

_tc2 = pl.pallas_call(
    _tc2_body,
    grid=_GRID,
    in_specs=[
        _rows_spec(128, lead=NC), _degp_spec(),
        _full((1, 128)), _full((128, 32)),
    ],
    out_specs=_rows_spec(32),
    out_shape=jax.ShapeDtypeStruct((N, 32), jnp.float32),
)

_tc3 = pl.pallas_call(
    _tc3_body,
    grid=_GRID,
    in_specs=[
        _rows_spec(32, lead=NC), _degp_spec(),
        _full((1, 32)), _full((32, 40)), _full((1, 40)),
    ],
    out_specs=_rows_spec(40),
    out_shape=jax.ShapeDtypeStruct((N, 40), jnp.float32),
)


def kernel(features, edge_index, W1, b1, W2, b2, W3, b3, Wl, bl):
  z16 = jnp.zeros((N, DEGW), jnp.float32)
  z128 = jnp.zeros((N, 128), jnp.float32)
  z32 = jnp.zeros((N, 32), jnp.float32)
  eint128 = edge_index.reshape(2, E // 128, 128).swapaxes(0, 1)
  eint512 = edge_index.reshape(2, E // 512, 512).swapaxes(0, 1)
  degp = _deg_kernel(eint128, z16)
  u1 = _tc0(degp, features)
  s1p = _prop128(u1, eint128, z128)
  u2 = _tc1(s1p, degp, W1, b1.reshape(1, -1), W2)
  s2p = _prop128(u2, eint128, z128)
  u3 = _tc2(s2p, degp, b2.reshape(1, -1), W3)
  s3p = _prop32(u3, eint512, z32)
  return _tc3(s3p, degp, b3.reshape(1, -1), Wl, bl.reshape(1, -1))


# speedup vs baseline: 1.0952x; 1.0001x over previous
import functools

import jax
import jax.numpy as jnp
from jax import lax
from jax.experimental import pallas as pl
from jax.experimental.pallas import tpu as pltpu
from jax.experimental.pallas import tpu_sc as plsc

N = 10000
E = 320000
NC = 2
NS = 16
NW = NC * NS
DEGW = 16
ZLEN = 624
LASTOFF = ZLEN * (NS - 1)
LASTLEN = N - LASTOFF

_mesh = lambda: plsc.VectorSubcoreMesh(core_axis_name="c", subcore_axis_name="s")


def _worker(c, s):
  return s * NC + c


def _tilewise_copy(s, make_src, make_dst):
  r0 = pl.multiple_of(s * ZLEN, 8)

  @pl.when(s < NS - 1)
  def _():
    pltpu.sync_copy(make_src(r0, ZLEN), make_dst(r0, ZLEN))

  @pl.when(s == NS - 1)
  def _():
    pltpu.sync_copy(make_src(LASTOFF, LASTLEN), make_dst(LASTOFF, LASTLEN))


DK = 128
DNCH = E // DK
DFLOOR = DNCH // NW
DREM = DNCH % NW
DMAXN = DFLOOR + 1
DR = 6


@functools.partial(
    pl.kernel,
    out_type=jax.ShapeDtypeStruct((NC, 2, N, DEGW), jnp.float32),
    mesh=_mesh(),
    compiler_params=pltpu.CompilerParams(use_tc_tiling_on_sc=False),
    scratch_types=[
        pltpu.VMEM((DR, 2, DK), jnp.int32),
        pltpu.VMEM((DK, DEGW), jnp.float32),
        pltpu.VMEM_SHARED((N, DEGW), jnp.float32),
        pltpu.VMEM_SHARED((N, DEGW), jnp.float32),
    ] + [pltpu.SemaphoreType.DMA] * (3 * DR),
)
def _deg_kernel(eint, zrows, out, ebuf, ones_v, acc_s, acc_d, *sems):
  isem = sems[0:DR]
  ssem = sems[DR:2 * DR]
  tsem = sems[2 * DR:3 * DR]
  c = lax.axis_index("c")
  s = lax.axis_index("s")
  w = _worker(c, s)
  _tilewise_copy(s, lambda o, l: zrows.at[pl.ds(o, l)],
                 lambda o, l: acc_s.at[pl.ds(o, l)])
  _tilewise_copy(s, lambda o, l: zrows.at[pl.ds(o, l)],
                 lambda o, l: acc_d.at[pl.ds(o, l)])

  def fill(i, carry):
    ones_v[i, :] = jnp.ones((DEGW,), jnp.float32)
    return carry

  lax.fori_loop(0, DK, fill, 0)
  plsc.subcore_barrier()

  n = jnp.where(w < DREM, DFLOOR + 1, DFLOOR)

  def start_idx(t, r):
    pltpu.async_copy(eint.at[w + NW * t], ebuf.at[r], isem[r])

  def wait_idx(r):
    pltpu.make_async_copy(eint.at[0], ebuf.at[r], isem[r]).wait()

  def start_scatter(r):
    pltpu.async_copy(ones_v, acc_s.at[ebuf.at[r, 0]], ssem[r], add=True)
    pltpu.async_copy(ones_v, acc_d.at[ebuf.at[r, 1]], tsem[r], add=True)

  def wait_scatter(r):
    pltpu.make_async_copy(ones_v, acc_s.at[ebuf.at[r, 0]], ssem[r]).wait()
    pltpu.make_async_copy(ones_v, acc_d.at[ebuf.at[r, 1]], tsem[r]).wait()

  for r0 in range(3):
    @pl.when(r0 < n)
    def _():
      start_idx(r0, r0)

  def ring_body(g, carry):
    for j in range(DR):
      t = DR * g + j
      r = j

      @pl.when(t < n)
      def _():
        wait_idx(r)
        start_scatter(r)
        r3 = (j + 3) % DR

        @pl.when(t >= 3)
        def _():
          wait_scatter(r3)

        @pl.when(t + 3 < n)
        def _():
          start_idx(t + 3, r3)

    return carry

  lax.fori_loop(0, (DMAXN + DR - 1) // DR, ring_body, 0)
  for k in range(1, 4):
    for r in range(DR):
      @pl.when((n - k) % DR == r)
      def _():
        @pl.when(n >= k)
        def _():
          wait_scatter(r)

  plsc.subcore_barrier()
  _tilewise_copy(s, lambda o, l: acc_s.at[pl.ds(o, l)],
                 lambda o, l: out.at[c, 0, pl.ds(o, l)])
  _tilewise_copy(s, lambda o, l: acc_d.at[pl.ds(o, l)],
                 lambda o, l: out.at[c, 1, pl.ds(o, l)])


def _make_prop(F, K, NB, L):
  SS = max(1, K // 128)
  NCH = E // K
  FLOOR = NCH // NW
  REM = NCH % NW
  MAXN = FLOOR + (1 if REM else 0)
  R = 2 * NB
  X = 4

  @functools.partial(
      pl.kernel,
      out_type=jax.ShapeDtypeStruct((NC, N, F), jnp.float32),
      mesh=_mesh(),
      compiler_params=pltpu.CompilerParams(use_tc_tiling_on_sc=False),
      scratch_types=[
          pltpu.VMEM((R, 2, K), jnp.int32),
          pltpu.VMEM((NB, K, F), jnp.float32),
          pltpu.VMEM_SHARED((N, F), jnp.float32),
      ] + [pltpu.SemaphoreType.DMA] * (R + 2 * NB),
  )
  def _prop(x, eint, zrows, out, ebuf, rows, acc, *sems):
    isem = sems[0:R]
    gsem = sems[R:R + NB]
    ssem = sems[R + NB:R + 2 * NB]
    c = lax.axis_index("c")
    s = lax.axis_index("s")
    w = _worker(c, s)
    _tilewise_copy(s, lambda o, l: zrows.at[pl.ds(o, l)],
                   lambda o, l: acc.at[pl.ds(o, l)])
    plsc.subcore_barrier()

    n = jnp.where(w < REM, FLOOR + 1, FLOOR)

    def start_idx(t, r):
      pltpu.async_copy(eint.at[w + NW * t], ebuf.at[r], isem[r])

    def wait_idx(r):
      pltpu.make_async_copy(eint.at[0], ebuf.at[r], isem[r]).wait()

    def start_gather(r, b):
      pltpu.async_copy(x.at[ebuf.at[r, 0]], rows.at[b], gsem[b])

    def wait_gather(r, b):
      pltpu.make_async_copy(x.at[ebuf.at[r, 0]], rows.at[b], gsem[b]).wait()

    def start_scatter(r, b):
      if SS == 1:
        pltpu.async_copy(rows.at[b], acc.at[ebuf.at[r, 1]], ssem[b], add=True)
      else:
        for j in range(SS):
          pltpu.async_copy(rows.at[b, pl.ds(128 * j, 128)],
                           acc.at[ebuf.at[r, 1, pl.ds(128 * j, 128)]],
                           ssem[b], add=True)

    def wait_scatter(b):
      if SS == 1:
        pltpu.make_async_copy(rows.at[b], acc.at[ebuf.at[0, 1]],
                              ssem[b]).wait()
      else:
        for j in range(SS):
          pltpu.make_async_copy(rows.at[b, pl.ds(128 * j, 128)],
                                acc.at[ebuf.at[0, 1, pl.ds(0, 128)]],
                                ssem[b]).wait()

    for t0 in range(X):
      @pl.when(t0 < n)
      def _():
        start_idx(t0, t0)

    for t0 in range(L):
      @pl.when(t0 < n)
      def _():
        wait_idx(t0)
        start_gather(t0, t0)

    def ring_body(g, carry):
      for j in range(R):
        t = R * g + j
        r = j
        b = j % NB
        bL = (b + L) % NB
        rL = (j + L) % R

        def lead_gather():
          @pl.when(t + L < n)
          def _():
            wait_idx(rL)
            start_gather(rL, bL)

        if L == 1:
          lead_gather()

        @pl.when(t < n)
        def _():
          wait_gather(r, b)
          start_scatter(r, b)

          @pl.when(t >= NB - 2)
          def _():
            wait_scatter((b + 2) % NB)

          if L >= 2:
            lead_gather()
          rx = (j + X) % R

          @pl.when(t + X < n)
          def _():
            start_idx(t + X, rx)

      return carry

    lax.fori_loop(0, (MAXN + R - 1) // R, ring_body, 0)
    for k in range(1, NB - 1):
      for b in range(NB):
        @pl.when((n - k) % NB == b)
        def _():
          @pl.when(n >= k)
          def _():
            wait_scatter(b)

    plsc.subcore_barrier()
    _tilewise_copy(s, lambda o, l: acc.at[pl.ds(o, l)],
                   lambda o, l: out.at[c, pl.ds(o, l)])

  return _prop


_prop128 = _make_prop(128, 128, 3, 1)
_prop32 = _make_prop(32, 512, 4, 2)


BT = 2000


def _norms(degp_ref):
  sdeg = degp_ref[0, 0, :, 0:1] + degp_ref[1, 0, :, 0:1]
  ddeg = degp_ref[0, 1, :, 0:1] + degp_ref[1, 1, :, 0:1]
  ns = lax.rsqrt(jnp.maximum(sdeg, 1.0))
  nd = lax.rsqrt(jnp.maximum(ddeg, 1.0))
  return ns, nd


def _tc0_body(degp_ref, f_ref, u1_ref):
  ns, _ = _norms(degp_ref)
  u1_ref[...] = f_ref[...] * ns


def _tc1_body(s1p_ref, degp_ref, w1_ref, b1_ref, w2_ref, u2_ref):
  ns, nd = _norms(degp_ref)
  s1 = (s1p_ref[0] + s1p_ref[1]) * nd
  h1 = jnp.maximum(
      jnp.dot(s1, w1_ref[...], preferred_element_type=jnp.float32) + b1_ref[...],
      0.0)
  u2_ref[...] = jnp.dot(h1 * ns, w2_ref[...], preferred_element_type=jnp.float32)


def _tc2_body(s2p_ref, degp_ref, b2_ref, w3_ref, u3_ref):
  ns, nd = _norms(degp_ref)
  h2 = jnp.maximum((s2p_ref[0] + s2p_ref[1]) * nd + b2_ref[...], 0.0)
  u3_ref[...] = jnp.dot(h2 * ns, w3_ref[...], preferred_element_type=jnp.float32)


def _tc3_body(s3p_ref, degp_ref, b3_ref, wl_ref, bl_ref, out_ref):
  _, nd = _norms(degp_ref)
  h3 = jnp.maximum((s3p_ref[0] + s3p_ref[1]) * nd + b3_ref[...], 0.0)
  out_ref[...] = (
      jnp.dot(h3, wl_ref[...], preferred_element_type=jnp.float32) + bl_ref[...])


def _degp_spec():
  return pl.BlockSpec((NC, 2, BT, DEGW), lambda i: (0, 0, i, 0))


def _full(shape):
  return pl.BlockSpec(shape, lambda i: tuple(0 for _ in shape))


def _rows_spec(F, lead=None):
  if lead is None:
    return pl.BlockSpec((BT, F), lambda i: (i, 0))
  return pl.BlockSpec((lead, BT, F), lambda i: (0, i, 0))


_GRID = (N // BT,)

_tc0 = pl.pallas_call(
    _tc0_body,
    grid=_GRID,
    in_specs=[_degp_spec(), _rows_spec(128)],
    out_specs=_rows_spec(128),
    out_shape=jax.ShapeDtypeStruct((N, 128), jnp.float32),
)

_tc1 = pl.pallas_call(
    _tc1_body,
    grid=_GRID,
    in_specs=[
        _rows_spec(128, lead=NC), _degp_spec(),
        _full((128, 512)), _full((1, 512)), _full((512, 128)),
    ],
    out_specs=_rows_spec(128),
    out_shape=jax.ShapeDtypeStruct((N, 128), jnp.float32),
)

_tc2 = pl.pallas_call(
    _tc2_body,
    grid=_GRID,
    in_specs=[
        _rows_spec(128, lead=NC), _degp_spec(),
        _full((1, 128)), _full((128, 32)),
    ],
    out_specs=_rows_spec(32),
    out_shape=jax.ShapeDtypeStruct((N, 32), jnp.float32),
)

_tc3 = pl.pallas_call(
    _tc3_body,
    grid=_GRID,
    in_specs=[
        _rows_spec(32, lead=NC), _degp_spec(),
        _full((1, 32)), _full((32, 40)), _full((1, 40)),
    ],
    out_specs=_rows_spec(40),
    out_shape=jax.ShapeDtypeStruct((N, 40), jnp.float32),
)


def kernel(features, edge_index, W1, b1, W2, b2, W3, b3, Wl, bl):
  z16 = jnp.zeros((N, DEGW), jnp.float32)
  z128 = jnp.zeros((N, 128), jnp.float32)
  z32 = jnp.zeros((N, 32), jnp.float32)
  eint128 = edge_index.reshape(2, E // 128, 128).swapaxes(0, 1)
  eint512 = edge_index.reshape(2, E // 512, 512).swapaxes(0, 1)
  degp = _deg_kernel(eint128, z16)
  u1 = _tc0(degp, features)
  s1p = _prop128(u1, eint128, z128)
  u2 = _tc1(s1p, degp, W1, b1.reshape(1, -1), W2)
  s2p = _prop128(u2, eint128, z128)
  u3 = _tc2(s2p, degp, b2.reshape(1, -1), W3)
  s3p = _prop32(u3, eint512, z32)
  return _tc3(s3p, degp, b3.reshape(1, -1), Wl, bl.reshape(1, -1))
